# baseline (speedup 1.0000x reference)
import functools

import jax
import jax.numpy as jnp
from jax import lax
from jax.experimental import pallas as pl
from jax.experimental.pallas import tpu as pltpu
from jax.experimental.pallas import tpu_sc as plsc

NS = 16
W_WIN = 80
CH = 8
BR = 640


def _npad(n):
    return -(-n // BR) * BR


_DN = (((1,), (1,)), ((), ()))


def _project_body(h_ref, wm_ref, ws_ref, out_ref):
    hb = h_ref[...].astype(jnp.bfloat16)
    out_ref[0] = (
        lax.dot_general(hb, wm_ref[0].astype(jnp.bfloat16), _DN,
                        preferred_element_type=jnp.float32)
        + lax.dot_general(hb, ws_ref[...].astype(jnp.bfloat16), _DN,
                          preferred_element_type=jnp.float32))


def _project(h, wm, wsk, npad):
    n, d = h.shape
    return pl.pallas_call(
        _project_body,
        grid=(2, npad // BR),
        in_specs=[
            pl.BlockSpec((BR, d), lambda c, g: (g, 0)),
            pl.BlockSpec((1, d, d), lambda c, g: (c, 0, 0)),
            pl.BlockSpec((d, d), lambda c, g: (0, 0)),
        ],
        out_specs=pl.BlockSpec((1, BR, d), lambda c, g: (c, g, 0)),
        out_shape=jax.ShapeDtypeStruct((2, npad, d), jnp.float32),
    )(h, wm, wsk)


def _post_body(h_ref, ai_ref, ao_ref, ci_ref, co_ref, bmi_ref, bsi_ref,
               bmo_ref, bso_ref, out_ref):
    ic = ai_ref[...] + bmi_ref[...] + bsi_ref[...]
    oc = ao_ref[...] + bmo_ref[...] + bso_ref[...]
    out_ref[...] = jnp.tanh(h_ref[...] + ci_ref[...] * ic + co_ref[...] * oc)


def _post(h, agg, ci, co, bmi, bsi, bmo, bso, npad):
    n, d = h.shape
    nb = npad // BR
    return pl.pallas_call(
        _post_body,
        grid=(nb,),
        in_specs=[
            pl.BlockSpec((BR, d), lambda g: (g, 0)),
            pl.BlockSpec((BR, d), lambda g: (g, 0)),
            pl.BlockSpec((BR, d), lambda g, _nb=nb: (g + _nb, 0)),
            pl.BlockSpec((BR, 1), lambda g: (g, 0)),
            pl.BlockSpec((BR, 1), lambda g: (g, 0)),
            pl.BlockSpec((1, d), lambda g: (0, 0)),
            pl.BlockSpec((1, d), lambda g: (0, 0)),
            pl.BlockSpec((1, d), lambda g: (0, 0)),
            pl.BlockSpec((1, d), lambda g: (0, 0)),
        ],
        out_specs=pl.BlockSpec((BR, d), lambda g: (g, 0)),
        out_shape=jax.ShapeDtypeStruct((n, d), jnp.float32),
    )(h, agg, agg, ci, co, bmi, bsi, bmo, bso)


def _decode_body(h_ref, ai_ref, ao_ref, ci_ref, co_ref, bmi_ref, bsi_ref,
                 bmo_ref, bso_ref, dw_ref, db_ref, logp_ref, finn_ref):
    ic = ai_ref[...] + bmi_ref[...] + bsi_ref[...]
    oc = ao_ref[...] + bmo_ref[...] + bso_ref[...]
    fin = jnp.tanh(h_ref[...] + ci_ref[...] * ic + co_ref[...] * oc)
    logits = lax.dot_general(
        fin.astype(jnp.bfloat16), dw_ref[...].astype(jnp.bfloat16), _DN,
        preferred_element_type=jnp.float32) + db_ref[...]
    m = jnp.max(logits, axis=-1, keepdims=True)
    lse = jnp.log(jnp.sum(jnp.exp(logits - m), axis=-1, keepdims=True)) + m
    logp_ref[...] = logits - lse
    nrm = jnp.sqrt(jnp.sum(fin * fin, axis=-1, keepdims=True))
    finn_ref[...] = fin / (nrm + 1e-12)


def _decode(h, agg, ci, co, bmi, bsi, bmo, bso, dw, db, npad):
    n, d = h.shape
    nc = dw.shape[0]
    nb = npad // BR
    return pl.pallas_call(
        _decode_body,
        grid=(nb,),
        in_specs=[
            pl.BlockSpec((BR, d), lambda g: (g, 0)),
            pl.BlockSpec((BR, d), lambda g: (g, 0)),
            pl.BlockSpec((BR, d), lambda g, _nb=nb: (g + _nb, 0)),
            pl.BlockSpec((BR, 1), lambda g: (g, 0)),
            pl.BlockSpec((BR, 1), lambda g: (g, 0)),
            pl.BlockSpec((1, d), lambda g: (0, 0)),
            pl.BlockSpec((1, d), lambda g: (0, 0)),
            pl.BlockSpec((1, d), lambda g: (0, 0)),
            pl.BlockSpec((1, d), lambda g: (0, 0)),
            pl.BlockSpec((nc, d), lambda g: (0, 0)),
            pl.BlockSpec((1, nc), lambda g: (0, 0)),
        ],
        out_specs=[
            pl.BlockSpec((BR, nc), lambda g: (g, 0)),
            pl.BlockSpec((BR, d), lambda g: (g, 0)),
        ],
        out_shape=[
            jax.ShapeDtypeStruct((n, nc), jnp.float32),
            jax.ShapeDtypeStruct((n, d), jnp.float32),
        ],
    )(h, agg, agg, ci, co, bmi, bsi, bmo, bso, dw, db)


NBUF = 4
CHW = 10
LA = 2


@functools.lru_cache(maxsize=None)
def _make_prop(npad, e, d):
    wpt = e // W_WIN // NS
    rpt = npad // NS
    assert wpt % CHW == 0
    mesh = plsc.VectorSubcoreMesh(core_axis_name="c", subcore_axis_name="s",
                                  num_cores=2, num_subcores=NS)

    @functools.partial(
        pl.kernel,
        out_type=jax.ShapeDtypeStruct((2 * npad, d), jnp.float32),
        mesh=mesh,
        scratch_types=[
            pltpu.VMEM((CHW, 2, W_WIN), jnp.int32),
            pltpu.VMEM((CHW, 1, W_WIN), jnp.float32),
            pltpu.VMEM((NBUF, W_WIN, d), jnp.float32),
            pltpu.VMEM_SHARED((npad, d), jnp.float32),
            pltpu.SemaphoreType.DMA((NBUF,)),
            pltpu.SemaphoreType.DMA((NBUF,)),
            pltpu.SemaphoreType.DMA,
        ],
    )
    def prop(xw, edges, ews, zeros, out, idx_v, ew_v, rows_v, acc, gsem,
             ssem, zsem):
        c = lax.axis_index("c")
        s = lax.axis_index("s")
        zdesc = pltpu.async_copy(zeros.at[pl.ds(s * rpt, rpt)],
                                 acc.at[pl.ds(s * rpt, rpt)], zsem)
        zdesc.wait()
        plsc.subcore_barrier()

        w_base = (c * NS + s) * wpt

        def scale(g, cc, _b=0, _w=0):
            ws = ew_v[_w, 0, pl.ds(g * 16, 16)]
            for l in range(16):
                eix = g * 16 + l
                sv = ws[l]
                for j in range(d // 16):
                    rows_v[_b, eix, pl.ds(j * 16, 16)] = (
                        rows_v[_b, eix, pl.ds(j * 16, 16)] * sv)
            return cc

        def run_scale(_b, _w):
            lax.fori_loop(0, W_WIN // 16,
                          functools.partial(scale, _b=_b, _w=_w), 0)

        def chunk_body(i, carry):
            r0 = w_base + i * CHW
            pltpu.sync_copy(edges.at[pl.ds(r0, CHW)], idx_v)
            pltpu.sync_copy(ews.at[pl.ds(r0, CHW)], ew_v)
            gds = [None] * CHW
            sds = [None] * CHW

            def fire_gather(w):
                b = w % NBUF
                if w >= NBUF:
                    sds[w - NBUF].wait()
                gds[w] = pltpu.async_copy(
                    xw.at[pl.ds(w * W_WIN, W_WIN)], rows_v.at[b],
                    gsem.at[b])

            for w in range(LA):
                fire_gather(w)
            for w in range(CHW):
                if w + LA < CHW:
                    fire_gather(w + LA)
                b = w % NBUF
                gds[w].wait()
                run_scale(b, w)
                sds[w] = pltpu.async_copy(
                    rows_v.at[b], acc.at[idx_v.at[w, 1]], ssem.at[b],
                    add=True)
            for w in range(CHW - NBUF, CHW):
                sds[w].wait()
            return carry

        lax.fori_loop(0, wpt // CHW, chunk_body, 0)
        plsc.subcore_barrier()
        pltpu.sync_copy(acc.at[pl.ds(s * rpt, rpt)],
                        out.at[pl.ds(c * npad + s * rpt, rpt)])

    return prop


def kernel(x, edge_index_in, edge_weight_in, edge_index_out, edge_weight_out,
           params):
    n, d = x.shape
    e = edge_index_in.shape[1]
    npad = _npad(n)

    srcs = jnp.concatenate(
        [edge_index_in[0], edge_index_out[0] + npad]).reshape(
            2 * e // W_WIN, W_WIN)
    dsts = jnp.concatenate(
        [edge_index_in[1], edge_index_out[1]]).reshape(2 * e // W_WIN, W_WIN)
    ews = jnp.concatenate(
        [edge_weight_in, edge_weight_out]).reshape(2 * e // W_WIN, 1, W_WIN)
    edges = jnp.stack([srcs, dsts], axis=1)
    zeros = jnp.zeros((npad, d), jnp.float32)

    prop = _make_prop(npad, e, d)

    h = x
    for li in (1, 2, 3):
        p = params[f'conv{li}']
        wm = jnp.stack([p['W_main_in'], p['W_main_out']])
        xw = _project(h, wm, p['W_skip'], npad)
        agg = prop(xw.reshape(2 * npad, d), edges, ews, zeros)
        args = (h, agg, p['C_in'], p['C_out'],
                p['b_main_in'].reshape(1, d), p['b_skip_in'].reshape(1, d),
                p['b_main_out'].reshape(1, d), p['b_skip_out'].reshape(1, d))
        if li < 3:
            h = _post(*args, npad)
        else:
            logp, finn = _decode(*args, params['dec_W'],
                                 params['dec_b'].reshape(1, -1), npad)
    return (logp, finn)


# speedup vs baseline: 1.0436x; 1.0436x over previous
import functools

import jax
import jax.numpy as jnp
from jax import lax
from jax.experimental import pallas as pl
from jax.experimental.pallas import tpu as pltpu
from jax.experimental.pallas import tpu_sc as plsc

NS = 16
W_WIN = 80
CH = 8
BR = 640


def _npad(n):
    return -(-n // BR) * BR


_DN = (((1,), (1,)), ((), ()))


def _project_body(h_ref, wm_ref, ws_ref, out_ref):
    hb = h_ref[...].astype(jnp.bfloat16)
    out_ref[0] = (
        lax.dot_general(hb, wm_ref[0].astype(jnp.bfloat16), _DN,
                        preferred_element_type=jnp.float32)
        + lax.dot_general(hb, ws_ref[...].astype(jnp.bfloat16), _DN,
                          preferred_element_type=jnp.float32))


def _project(h, wm, wsk, npad):
    n, d = h.shape
    return pl.pallas_call(
        _project_body,
        grid=(2, npad // BR),
        in_specs=[
            pl.BlockSpec((BR, d), lambda c, g: (g, 0)),
            pl.BlockSpec((1, d, d), lambda c, g: (c, 0, 0)),
            pl.BlockSpec((d, d), lambda c, g: (0, 0)),
        ],
        out_specs=pl.BlockSpec((1, BR, d), lambda c, g: (c, g, 0)),
        out_shape=jax.ShapeDtypeStruct((2, npad, d), jnp.float32),
    )(h, wm, wsk)


def _post_body(h_ref, ai_ref, ao_ref, ci_ref, co_ref, bmi_ref, bsi_ref,
               bmo_ref, bso_ref, out_ref):
    ic = ai_ref[...] + bmi_ref[...] + bsi_ref[...]
    oc = ao_ref[...] + bmo_ref[...] + bso_ref[...]
    out_ref[...] = jnp.tanh(h_ref[...] + ci_ref[...] * ic + co_ref[...] * oc)


def _post(h, agg, ci, co, bmi, bsi, bmo, bso, npad):
    n, d = h.shape
    nb = npad // BR
    return pl.pallas_call(
        _post_body,
        grid=(nb,),
        in_specs=[
            pl.BlockSpec((BR, d), lambda g: (g, 0)),
            pl.BlockSpec((BR, d), lambda g: (g, 0)),
            pl.BlockSpec((BR, d), lambda g, _nb=nb: (g + _nb, 0)),
            pl.BlockSpec((BR, 1), lambda g: (g, 0)),
            pl.BlockSpec((BR, 1), lambda g: (g, 0)),
            pl.BlockSpec((1, d), lambda g: (0, 0)),
            pl.BlockSpec((1, d), lambda g: (0, 0)),
            pl.BlockSpec((1, d), lambda g: (0, 0)),
            pl.BlockSpec((1, d), lambda g: (0, 0)),
        ],
        out_specs=pl.BlockSpec((BR, d), lambda g: (g, 0)),
        out_shape=jax.ShapeDtypeStruct((n, d), jnp.float32),
    )(h, agg, agg, ci, co, bmi, bsi, bmo, bso)


def _postproj_body(h_ref, ai_ref, ao_ref, ci_ref, co_ref, bmi_ref, bsi_ref,
                   bmo_ref, bso_ref, wm_ref, ws_ref, hn_ref, xw_ref):
    ic = ai_ref[...] + bmi_ref[...] + bsi_ref[...]
    oc = ao_ref[...] + bmo_ref[...] + bso_ref[...]
    hn = jnp.tanh(h_ref[...] + ci_ref[...] * ic + co_ref[...] * oc)
    hn_ref[...] = hn
    hb = hn.astype(jnp.bfloat16)
    wsb = ws_ref[...].astype(jnp.bfloat16)
    xw_ref[0] = (
        lax.dot_general(hb, wm_ref[0].astype(jnp.bfloat16), _DN,
                        preferred_element_type=jnp.float32)
        + lax.dot_general(hb, wsb, _DN, preferred_element_type=jnp.float32))
    xw_ref[1] = (
        lax.dot_general(hb, wm_ref[1].astype(jnp.bfloat16), _DN,
                        preferred_element_type=jnp.float32)
        + lax.dot_general(hb, wsb, _DN, preferred_element_type=jnp.float32))


def _postproj(h, agg, ci, co, bmi, bsi, bmo, bso, wm, wsk, npad):
    n, d = h.shape
    nb = npad // BR
    return pl.pallas_call(
        _postproj_body,
        grid=(nb,),
        in_specs=[
            pl.BlockSpec((BR, d), lambda g: (g, 0)),
            pl.BlockSpec((BR, d), lambda g: (g, 0)),
            pl.BlockSpec((BR, d), lambda g, _nb=nb: (g + _nb, 0)),
            pl.BlockSpec((BR, 1), lambda g: (g, 0)),
            pl.BlockSpec((BR, 1), lambda g: (g, 0)),
            pl.BlockSpec((1, d), lambda g: (0, 0)),
            pl.BlockSpec((1, d), lambda g: (0, 0)),
            pl.BlockSpec((1, d), lambda g: (0, 0)),
            pl.BlockSpec((1, d), lambda g: (0, 0)),
            pl.BlockSpec((2, d, d), lambda g: (0, 0, 0)),
            pl.BlockSpec((d, d), lambda g: (0, 0)),
        ],
        out_specs=[
            pl.BlockSpec((BR, d), lambda g: (g, 0)),
            pl.BlockSpec((2, BR, d), lambda g: (0, g, 0)),
        ],
        out_shape=[
            jax.ShapeDtypeStruct((n, d), jnp.float32),
            jax.ShapeDtypeStruct((2, npad, d), jnp.float32),
        ],
    )(h, agg, agg, ci, co, bmi, bsi, bmo, bso, wm, wsk)


def _decode_body(h_ref, ai_ref, ao_ref, ci_ref, co_ref, bmi_ref, bsi_ref,
                 bmo_ref, bso_ref, dw_ref, db_ref, logp_ref, finn_ref):
    ic = ai_ref[...] + bmi_ref[...] + bsi_ref[...]
    oc = ao_ref[...] + bmo_ref[...] + bso_ref[...]
    fin = jnp.tanh(h_ref[...] + ci_ref[...] * ic + co_ref[...] * oc)
    logits = lax.dot_general(
        fin.astype(jnp.bfloat16), dw_ref[...].astype(jnp.bfloat16), _DN,
        preferred_element_type=jnp.float32) + db_ref[...]
    m = jnp.max(logits, axis=-1, keepdims=True)
    lse = jnp.log(jnp.sum(jnp.exp(logits - m), axis=-1, keepdims=True)) + m
    logp_ref[...] = logits - lse
    nrm = jnp.sqrt(jnp.sum(fin * fin, axis=-1, keepdims=True))
    finn_ref[...] = fin / (nrm + 1e-12)


def _decode(h, agg, ci, co, bmi, bsi, bmo, bso, dw, db, npad):
    n, d = h.shape
    nc = dw.shape[0]
    nb = npad // BR
    return pl.pallas_call(
        _decode_body,
        grid=(nb,),
        in_specs=[
            pl.BlockSpec((BR, d), lambda g: (g, 0)),
            pl.BlockSpec((BR, d), lambda g: (g, 0)),
            pl.BlockSpec((BR, d), lambda g, _nb=nb: (g + _nb, 0)),
            pl.BlockSpec((BR, 1), lambda g: (g, 0)),
            pl.BlockSpec((BR, 1), lambda g: (g, 0)),
            pl.BlockSpec((1, d), lambda g: (0, 0)),
            pl.BlockSpec((1, d), lambda g: (0, 0)),
            pl.BlockSpec((1, d), lambda g: (0, 0)),
            pl.BlockSpec((1, d), lambda g: (0, 0)),
            pl.BlockSpec((nc, d), lambda g: (0, 0)),
            pl.BlockSpec((1, nc), lambda g: (0, 0)),
        ],
        out_specs=[
            pl.BlockSpec((BR, nc), lambda g: (g, 0)),
            pl.BlockSpec((BR, d), lambda g: (g, 0)),
        ],
        out_shape=[
            jax.ShapeDtypeStruct((n, nc), jnp.float32),
            jax.ShapeDtypeStruct((n, d), jnp.float32),
        ],
    )(h, agg, agg, ci, co, bmi, bsi, bmo, bso, dw, db)


NBUF = 4
CHW = 10
LA = 2


@functools.lru_cache(maxsize=None)
def _make_prop(npad, e, d):
    wpt = e // W_WIN // NS
    rpt = npad // NS
    assert wpt % CHW == 0
    mesh = plsc.VectorSubcoreMesh(core_axis_name="c", subcore_axis_name="s",
                                  num_cores=2, num_subcores=NS)

    @functools.partial(
        pl.kernel,
        out_type=jax.ShapeDtypeStruct((2 * npad, d), jnp.float32),
        mesh=mesh,
        scratch_types=[
            pltpu.VMEM((CHW, 2, W_WIN), jnp.int32),
            pltpu.VMEM((CHW, 1, W_WIN), jnp.float32),
            pltpu.VMEM((NBUF, W_WIN, d), jnp.float32),
            pltpu.VMEM_SHARED((npad, d), jnp.float32),
            pltpu.SemaphoreType.DMA((NBUF,)),
            pltpu.SemaphoreType.DMA((NBUF,)),
            pltpu.SemaphoreType.DMA,
        ],
    )
    def prop(xw, edges, ews, zeros, out, idx_v, ew_v, rows_v, acc, gsem,
             ssem, zsem):
        c = lax.axis_index("c")
        s = lax.axis_index("s")
        zdesc = pltpu.async_copy(zeros.at[pl.ds(s * rpt, rpt)],
                                 acc.at[pl.ds(s * rpt, rpt)], zsem)
        zdesc.wait()
        plsc.subcore_barrier()

        w_base = (c * NS + s) * wpt

        def scale(g, cc, _b=0, _w=0):
            ws = ew_v[_w, 0, pl.ds(g * 16, 16)]
            for l in range(16):
                eix = g * 16 + l
                sv = ws[l]
                for j in range(d // 16):
                    rows_v[_b, eix, pl.ds(j * 16, 16)] = (
                        rows_v[_b, eix, pl.ds(j * 16, 16)] * sv)
            return cc

        def run_scale(_b, _w):
            lax.fori_loop(0, W_WIN // 16,
                          functools.partial(scale, _b=_b, _w=_w), 0)

        def chunk_body(i, carry):
            r0 = w_base + i * CHW
            pltpu.sync_copy(edges.at[pl.ds(r0, CHW)], idx_v)
            pltpu.sync_copy(ews.at[pl.ds(r0, CHW)], ew_v)
            gds = [None] * CHW
            sds = [None] * CHW

            def fire_gather(w):
                b = w % NBUF
                if w >= NBUF:
                    sds[w - NBUF].wait()
                gds[w] = pltpu.async_copy(
                    xw.at[idx_v.at[w, 0]], rows_v.at[b], gsem.at[b])

            for w in range(LA):
                fire_gather(w)
            for w in range(CHW):
                if w + LA < CHW:
                    fire_gather(w + LA)
                b = w % NBUF
                gds[w].wait()
                run_scale(b, w)
                sds[w] = pltpu.async_copy(
                    rows_v.at[b], acc.at[idx_v.at[w, 1]], ssem.at[b],
                    add=True)
            for w in range(CHW - NBUF, CHW):
                sds[w].wait()
            return carry

        lax.fori_loop(0, wpt // CHW, chunk_body, 0)
        plsc.subcore_barrier()
        pltpu.sync_copy(acc.at[pl.ds(s * rpt, rpt)],
                        out.at[pl.ds(c * npad + s * rpt, rpt)])

    return prop


def kernel(x, edge_index_in, edge_weight_in, edge_index_out, edge_weight_out,
           params):
    n, d = x.shape
    e = edge_index_in.shape[1]
    npad = _npad(n)

    srcs = jnp.concatenate(
        [edge_index_in[0], edge_index_out[0] + npad]).reshape(
            2 * e // W_WIN, W_WIN)
    dsts = jnp.concatenate(
        [edge_index_in[1], edge_index_out[1]]).reshape(2 * e // W_WIN, W_WIN)
    ews = jnp.concatenate(
        [edge_weight_in, edge_weight_out]).reshape(2 * e // W_WIN, 1, W_WIN)
    edges = jnp.stack([srcs, dsts], axis=1)
    zeros = jnp.zeros((npad, d), jnp.float32)

    prop = _make_prop(npad, e, d)

    h = x
    p = params['conv1']
    wm = jnp.stack([p['W_main_in'], p['W_main_out']])
    xw = _project(h, wm, p['W_skip'], npad)
    for li in (1, 2, 3):
        p = params[f'conv{li}']
        agg = prop(xw.reshape(2 * npad, d), edges, ews, zeros)
        args = (h, agg, p['C_in'], p['C_out'],
                p['b_main_in'].reshape(1, d), p['b_skip_in'].reshape(1, d),
                p['b_main_out'].reshape(1, d), p['b_skip_out'].reshape(1, d))
        if li < 3:
            pn = params[f'conv{li + 1}']
            wmn = jnp.stack([pn['W_main_in'], pn['W_main_out']])
            h, xw = _postproj(*args, wmn, pn['W_skip'], npad)
        else:
            logp, finn = _decode(*args, params['dec_W'],
                                 params['dec_b'].reshape(1, -1), npad)
    return (logp, finn)


# speedup vs baseline: 1.1372x; 1.0898x over previous
import functools

import jax
import jax.numpy as jnp
from jax import lax
from jax.experimental import pallas as pl
from jax.experimental.pallas import tpu as pltpu
from jax.experimental.pallas import tpu_sc as plsc

NS = 16
W_WIN = 80
CH = 8
BR = 640


def _npad(n):
    return -(-n // BR) * BR


_DN = (((1,), (1,)), ((), ()))


def _project_body(h_ref, wm_ref, ws_ref, out_ref):
    hb = h_ref[...].astype(jnp.bfloat16)
    out_ref[0] = (
        lax.dot_general(hb, wm_ref[0].astype(jnp.bfloat16), _DN,
                        preferred_element_type=jnp.float32)
        + lax.dot_general(hb, ws_ref[...].astype(jnp.bfloat16), _DN,
                          preferred_element_type=jnp.float32))


def _project(h, wm, wsk, npad):
    n, d = h.shape
    return pl.pallas_call(
        _project_body,
        grid=(2, npad // BR),
        in_specs=[
            pl.BlockSpec((BR, d), lambda c, g: (g, 0)),
            pl.BlockSpec((1, d, d), lambda c, g: (c, 0, 0)),
            pl.BlockSpec((d, d), lambda c, g: (0, 0)),
        ],
        out_specs=pl.BlockSpec((1, BR, d), lambda c, g: (c, g, 0)),
        out_shape=jax.ShapeDtypeStruct((2, npad, d), jnp.float32),
    )(h, wm, wsk)


def _post_body(h_ref, ai_ref, ao_ref, ci_ref, co_ref, bmi_ref, bsi_ref,
               bmo_ref, bso_ref, out_ref):
    ic = ai_ref[...] + bmi_ref[...] + bsi_ref[...]
    oc = ao_ref[...] + bmo_ref[...] + bso_ref[...]
    out_ref[...] = jnp.tanh(h_ref[...] + ci_ref[...] * ic + co_ref[...] * oc)


def _post(h, agg, ci, co, bmi, bsi, bmo, bso, npad):
    n, d = h.shape
    nb = npad // BR
    return pl.pallas_call(
        _post_body,
        grid=(nb,),
        in_specs=[
            pl.BlockSpec((BR, d), lambda g: (g, 0)),
            pl.BlockSpec((BR, d), lambda g: (g, 0)),
            pl.BlockSpec((BR, d), lambda g, _nb=nb: (g + _nb, 0)),
            pl.BlockSpec((BR, 1), lambda g: (g, 0)),
            pl.BlockSpec((BR, 1), lambda g: (g, 0)),
            pl.BlockSpec((1, d), lambda g: (0, 0)),
            pl.BlockSpec((1, d), lambda g: (0, 0)),
            pl.BlockSpec((1, d), lambda g: (0, 0)),
            pl.BlockSpec((1, d), lambda g: (0, 0)),
        ],
        out_specs=pl.BlockSpec((BR, d), lambda g: (g, 0)),
        out_shape=jax.ShapeDtypeStruct((n, d), jnp.float32),
    )(h, agg, agg, ci, co, bmi, bsi, bmo, bso)


def _postproj_body(h_ref, ai_ref, ao_ref, ci_ref, co_ref, bmi_ref, bsi_ref,
                   bmo_ref, bso_ref, wm_ref, ws_ref, hn_ref, xw_ref):
    ic = ai_ref[...] + bmi_ref[...] + bsi_ref[...]
    oc = ao_ref[...] + bmo_ref[...] + bso_ref[...]
    hn = jnp.tanh(h_ref[...] + ci_ref[...] * ic + co_ref[...] * oc)
    hn_ref[...] = hn
    hb = hn.astype(jnp.bfloat16)
    wsb = ws_ref[...].astype(jnp.bfloat16)
    xw_ref[0] = (
        lax.dot_general(hb, wm_ref[0].astype(jnp.bfloat16), _DN,
                        preferred_element_type=jnp.float32)
        + lax.dot_general(hb, wsb, _DN, preferred_element_type=jnp.float32))
    xw_ref[1] = (
        lax.dot_general(hb, wm_ref[1].astype(jnp.bfloat16), _DN,
                        preferred_element_type=jnp.float32)
        + lax.dot_general(hb, wsb, _DN, preferred_element_type=jnp.float32))


def _postproj(h, agg, ci, co, bmi, bsi, bmo, bso, wm, wsk, npad):
    n, d = h.shape
    nb = npad // BR
    return pl.pallas_call(
        _postproj_body,
        grid=(nb,),
        in_specs=[
            pl.BlockSpec((BR, d), lambda g: (g, 0)),
            pl.BlockSpec((BR, d), lambda g: (g, 0)),
            pl.BlockSpec((BR, d), lambda g, _nb=nb: (g + _nb, 0)),
            pl.BlockSpec((BR, 1), lambda g: (g, 0)),
            pl.BlockSpec((BR, 1), lambda g: (g, 0)),
            pl.BlockSpec((1, d), lambda g: (0, 0)),
            pl.BlockSpec((1, d), lambda g: (0, 0)),
            pl.BlockSpec((1, d), lambda g: (0, 0)),
            pl.BlockSpec((1, d), lambda g: (0, 0)),
            pl.BlockSpec((2, d, d), lambda g: (0, 0, 0)),
            pl.BlockSpec((d, d), lambda g: (0, 0)),
        ],
        out_specs=[
            pl.BlockSpec((BR, d), lambda g: (g, 0)),
            pl.BlockSpec((2, BR, d), lambda g: (0, g, 0)),
        ],
        out_shape=[
            jax.ShapeDtypeStruct((n, d), jnp.float32),
            jax.ShapeDtypeStruct((2, npad, d), jnp.float32),
        ],
    )(h, agg, agg, ci, co, bmi, bsi, bmo, bso, wm, wsk)


def _decode_body(h_ref, ai_ref, ao_ref, ci_ref, co_ref, bmi_ref, bsi_ref,
                 bmo_ref, bso_ref, dw_ref, db_ref, logp_ref, finn_ref):
    ic = ai_ref[...] + bmi_ref[...] + bsi_ref[...]
    oc = ao_ref[...] + bmo_ref[...] + bso_ref[...]
    fin = jnp.tanh(h_ref[...] + ci_ref[...] * ic + co_ref[...] * oc)
    logits = lax.dot_general(
        fin.astype(jnp.bfloat16), dw_ref[...].astype(jnp.bfloat16), _DN,
        preferred_element_type=jnp.float32) + db_ref[...]
    m = jnp.max(logits, axis=-1, keepdims=True)
    lse = jnp.log(jnp.sum(jnp.exp(logits - m), axis=-1, keepdims=True)) + m
    logp_ref[...] = logits - lse
    nrm = jnp.sqrt(jnp.sum(fin * fin, axis=-1, keepdims=True))
    finn_ref[...] = fin / (nrm + 1e-12)


def _decode(h, agg, ci, co, bmi, bsi, bmo, bso, dw, db, npad):
    n, d = h.shape
    nc = dw.shape[0]
    nb = npad // BR
    return pl.pallas_call(
        _decode_body,
        grid=(nb,),
        in_specs=[
            pl.BlockSpec((BR, d), lambda g: (g, 0)),
            pl.BlockSpec((BR, d), lambda g: (g, 0)),
            pl.BlockSpec((BR, d), lambda g, _nb=nb: (g + _nb, 0)),
            pl.BlockSpec((BR, 1), lambda g: (g, 0)),
            pl.BlockSpec((BR, 1), lambda g: (g, 0)),
            pl.BlockSpec((1, d), lambda g: (0, 0)),
            pl.BlockSpec((1, d), lambda g: (0, 0)),
            pl.BlockSpec((1, d), lambda g: (0, 0)),
            pl.BlockSpec((1, d), lambda g: (0, 0)),
            pl.BlockSpec((nc, d), lambda g: (0, 0)),
            pl.BlockSpec((1, nc), lambda g: (0, 0)),
        ],
        out_specs=[
            pl.BlockSpec((BR, nc), lambda g: (g, 0)),
            pl.BlockSpec((BR, d), lambda g: (g, 0)),
        ],
        out_shape=[
            jax.ShapeDtypeStruct((n, nc), jnp.float32),
            jax.ShapeDtypeStruct((n, d), jnp.float32),
        ],
    )(h, agg, agg, ci, co, bmi, bsi, bmo, bso, dw, db)


NBUF = 4
CHW = 10
LA = 2


@functools.lru_cache(maxsize=None)
def _make_prop(npad, e, d):
    wpt = e // W_WIN // NS
    rpt = npad // NS
    assert wpt % CHW == 0
    mesh = plsc.VectorSubcoreMesh(core_axis_name="c", subcore_axis_name="s",
                                  num_cores=2, num_subcores=NS)

    @functools.partial(
        pl.kernel,
        out_type=jax.ShapeDtypeStruct((2 * npad, d), jnp.float32),
        mesh=mesh,
        scratch_types=[
            pltpu.VMEM((2, CHW, 2, W_WIN), jnp.int32),
            pltpu.VMEM((2, CHW, 1, W_WIN), jnp.float32),
            pltpu.VMEM((NBUF, W_WIN, d), jnp.float32),
            pltpu.VMEM_SHARED((npad, d), jnp.float32),
            pltpu.SemaphoreType.DMA((NBUF,)),
            pltpu.SemaphoreType.DMA((NBUF,)),
            pltpu.SemaphoreType.DMA((2,)),
            pltpu.SemaphoreType.DMA((2,)),
            pltpu.SemaphoreType.DMA,
        ],
    )
    def prop(xw, edges, ews, zeros, out, idx_v, ew_v, rows_v, acc, gsem,
             ssem, isem, wsem, zsem):
        c = lax.axis_index("c")
        s = lax.axis_index("s")
        zdesc = pltpu.async_copy(zeros.at[pl.ds(s * rpt, rpt)],
                                 acc.at[pl.ds(s * rpt, rpt)], zsem)
        zdesc.wait()
        plsc.subcore_barrier()

        w_base = (c * NS + s) * wpt

        def stage(ch, p):
            pltpu.async_copy(edges.at[pl.ds(w_base + ch * CHW, CHW)],
                             idx_v.at[p], isem.at[p])
            pltpu.async_copy(ews.at[pl.ds(w_base + ch * CHW, CHW)],
                             ew_v.at[p], wsem.at[p])

        def wait_stage(p):
            pltpu.make_async_copy(edges.at[pl.ds(0, CHW)], idx_v.at[p],
                                  isem.at[p]).wait()
            pltpu.make_async_copy(ews.at[pl.ds(0, CHW)], ew_v.at[p],
                                  wsem.at[p]).wait()

        def scale(g, cc, _b=0, _w=0, _p=0):
            ws = ew_v[_p, _w, 0, pl.ds(g * 16, 16)]
            for l in range(16):
                eix = g * 16 + l
                sv = ws[l]
                for j in range(d // 16):
                    rows_v[_b, eix, pl.ds(j * 16, 16)] = (
                        rows_v[_b, eix, pl.ds(j * 16, 16)] * sv)
            return cc

        def run_scale(_b, _w, _p):
            lax.fori_loop(0, W_WIN // 16,
                          functools.partial(scale, _b=_b, _w=_w, _p=_p), 0)

        nch = wpt // CHW

        def chunk_body(i, carry):
            p = i % 2
            @pl.when(i + 1 < nch)
            def _pf():
                stage(i + 1, 1 - p)
            wait_stage(p)
            gds = [None] * CHW
            sds = [None] * CHW

            def fire_gather(w):
                b = w % NBUF
                if w >= NBUF:
                    sds[w - NBUF].wait()
                gds[w] = pltpu.async_copy(
                    xw.at[idx_v.at[p, w, 0]], rows_v.at[b], gsem.at[b])

            for w in range(LA):
                fire_gather(w)
            for w in range(CHW):
                if w + LA < CHW:
                    fire_gather(w + LA)
                b = w % NBUF
                gds[w].wait()
                run_scale(b, w, p)
                sds[w] = pltpu.async_copy(
                    rows_v.at[b], acc.at[idx_v.at[p, w, 1]], ssem.at[b],
                    add=True)
            for w in range(CHW - NBUF, CHW):
                sds[w].wait()
            return carry

        stage(0, 0)
        lax.fori_loop(0, nch, chunk_body, 0)
        plsc.subcore_barrier()
        pltpu.sync_copy(acc.at[pl.ds(s * rpt, rpt)],
                        out.at[pl.ds(c * npad + s * rpt, rpt)])

    return prop


def kernel(x, edge_index_in, edge_weight_in, edge_index_out, edge_weight_out,
           params):
    n, d = x.shape
    e = edge_index_in.shape[1]
    npad = _npad(n)

    srcs = jnp.concatenate(
        [edge_index_in[0], edge_index_out[0] + npad]).reshape(
            2 * e // W_WIN, W_WIN)
    dsts = jnp.concatenate(
        [edge_index_in[1], edge_index_out[1]]).reshape(2 * e // W_WIN, W_WIN)
    ews = jnp.concatenate(
        [edge_weight_in, edge_weight_out]).reshape(2 * e // W_WIN, 1, W_WIN)
    edges = jnp.stack([srcs, dsts], axis=1)
    zeros = jnp.zeros((npad, d), jnp.float32)

    prop = _make_prop(npad, e, d)

    h = x
    p = params['conv1']
    wm = jnp.stack([p['W_main_in'], p['W_main_out']])
    xw = _project(h, wm, p['W_skip'], npad)
    for li in (1, 2, 3):
        p = params[f'conv{li}']
        agg = prop(xw.reshape(2 * npad, d), edges, ews, zeros)
        args = (h, agg, p['C_in'], p['C_out'],
                p['b_main_in'].reshape(1, d), p['b_skip_in'].reshape(1, d),
                p['b_main_out'].reshape(1, d), p['b_skip_out'].reshape(1, d))
        if li < 3:
            pn = params[f'conv{li + 1}']
            wmn = jnp.stack([pn['W_main_in'], pn['W_main_out']])
            h, xw = _postproj(*args, wmn, pn['W_skip'], npad)
        else:
            logp, finn = _decode(*args, params['dec_W'],
                                 params['dec_b'].reshape(1, -1), npad)
    return (logp, finn)
